# baseline (speedup 1.0000x reference)
import functools

import jax
import jax.numpy as jnp
from jax import lax
from jax.experimental import pallas as pl
from jax.experimental.pallas import tpu as pltpu

B = 1024
NP1 = 2001
D = 4
C = 8
K = 16
NEG = -1e9


def _row_kernel(cn_ref, psi_ref, psicur_ref, coords_ref, ccur_ref, knn_ref,
                dem_ref, vis_ref, uc_ref, wq_ref, bq_ref, lam_ref, mu_ref,
                sf_ref, logp_ref, mask_ref):
    b = pl.program_id(0)
    cn = cn_ref[b]
    remaining = 1.0 - uc_ref[0, 0, 0]

    one = jnp.float32(1.0)
    zero = jnp.float32(0.0)
    dem = dem_ref[0]
    vis = vis_ref[0]
    exceeds = jnp.where(dem > remaining, one, zero)
    maskf = jnp.maximum(vis, exceeds)
    iota = lax.broadcasted_iota(jnp.int32, (NP1, 1), 0)
    is_depot_pos = iota == 0
    at_depot_f = jnp.where(cn == 0, one, zero)
    open_cust = jnp.where((maskf < 0.5) & (iota > 0), one, zero)
    has_cust_f = jnp.max(open_cust)
    depot_val = at_depot_f * has_cust_f
    maskf = jnp.where(is_depot_pos, depot_val, maskf)
    mask_ref[0] = maskf

    cur_emb = psicur_ref[0, 0]
    cur_xy = ccur_ref[0, 0]
    wq = wq_ref[...]
    q = (cur_emb @ wq[0:4, :] + cur_xy @ wq[4:6, :]
         + remaining * wq[6:7, :] + sf_ref[0, 0] * wq[7:8, :] + bq_ref[...])

    psi = psi_ref[0]
    attn = jnp.dot(psi, q.reshape(D, 1),
                   preferred_element_type=jnp.float32) / jnp.sqrt(
                       jnp.float32(D))
    diff = coords_ref[0] - cur_xy
    dist = jnp.sqrt(jnp.sum(diff * diff, axis=1, keepdims=True) + 1e-10)
    hybrid = lam_ref[0, 0] * attn - mu_ref[0, 0] * dist

    knnv = knn_ref[0, 0]
    allowed = jnp.any(iota == knnv, axis=1, keepdims=True) | is_depot_pos
    scores = jnp.where((maskf > 0.5) | (~allowed), jnp.float32(NEG), hybrid)
    m = jnp.max(scores)
    sh = scores - m
    logp_ref[0] = sh - jnp.log(jnp.sum(jnp.exp(sh)))


def kernel(psi_prime, knn_indices, coords, demands, visited, current_node,
           used_capacity, W_q, b_q, lam, mu, step, n_customers):
    f32 = jnp.float32
    dem3 = demands.reshape(B, NP1, 1)
    vis3 = visited.astype(f32).reshape(B, NP1, 1)
    uc3 = used_capacity.astype(f32).reshape(B, 1, 1)
    cn = current_node.astype(jnp.int32)
    sf = (jnp.asarray(step, f32) / jnp.asarray(n_customers, f32)).reshape(1, 1)
    lam2 = jnp.asarray(lam, f32).reshape(1, 1)
    mu2 = jnp.asarray(mu, f32).reshape(1, 1)
    bq2 = b_q.reshape(1, D)

    grid_spec = pltpu.PrefetchScalarGridSpec(
        num_scalar_prefetch=1,
        grid=(B,),
        in_specs=[
            pl.BlockSpec((1, NP1, D), lambda b, cn_: (b, 0, 0)),
            pl.BlockSpec((1, 1, 1, D), lambda b, cn_: (b, cn_[b], 0, 0)),
            pl.BlockSpec((1, NP1, 2), lambda b, cn_: (b, 0, 0)),
            pl.BlockSpec((1, 1, 1, 2), lambda b, cn_: (b, cn_[b], 0, 0)),
            pl.BlockSpec((1, 1, 1, K), lambda b, cn_: (b, cn_[b], 0, 0)),
            pl.BlockSpec((1, NP1, 1), lambda b, cn_: (b, 0, 0)),
            pl.BlockSpec((1, NP1, 1), lambda b, cn_: (b, 0, 0)),
            pl.BlockSpec((1, 1, 1), lambda b, cn_: (b, 0, 0)),
            pl.BlockSpec((C, D), lambda b, cn_: (0, 0)),
            pl.BlockSpec((1, D), lambda b, cn_: (0, 0)),
            pl.BlockSpec((1, 1), lambda b, cn_: (0, 0)),
            pl.BlockSpec((1, 1), lambda b, cn_: (0, 0)),
            pl.BlockSpec((1, 1), lambda b, cn_: (0, 0)),
        ],
        out_specs=[
            pl.BlockSpec((1, NP1, 1), lambda b, cn_: (b, 0, 0)),
            pl.BlockSpec((1, NP1, 1), lambda b, cn_: (b, 0, 0)),
        ],
    )
    logp3, mask3 = pl.pallas_call(
        _row_kernel,
        grid_spec=grid_spec,
        out_shape=[
            jax.ShapeDtypeStruct((B, NP1, 1), f32),
            jax.ShapeDtypeStruct((B, NP1, 1), f32),
        ],
    )(cn, psi_prime, psi_prime.reshape(B, NP1, 1, D), coords,
      coords.reshape(B, NP1, 1, 2), knn_indices.reshape(B, NP1, 1, K),
      dem3, vis3, uc3, W_q, bq2, lam2, mu2, sf)
    return logp3.reshape(B, NP1), (mask3.reshape(B, NP1) > 0.5)
